# baseline (speedup 1.0000x reference)
import jax
import jax.numpy as jnp
from jax.experimental import pallas as pl
from jax.experimental.pallas import tpu as pltpu

_DELTA_AGG = 0.5
_NUM_IDS = 8


def _body(sv_ref, tt_ref, tk_ref, out_ref):
    sv = sv_ref[0]
    tt = tt_ref[0]
    tk = tk_ref[0]

    loss_sum = jnp.float32(0.0)
    valid_cnt = jnp.float32(0.0)
    for tid in range(1, _NUM_IDS):
        kif = (tk == tid).astype(jnp.float32)
        tif = (tt == tid).astype(jnp.float32)
        kcnt = jnp.sum(kif)
        tcnt = jnp.sum(tif)
        inv_k = 1.0 / jnp.maximum(kcnt, 1.0)
        sumsq = jnp.zeros((128, 128), jnp.float32)
        for c in range(4):
            g_c = jnp.sum(sv[c] * kif) * inv_k
            d_c = sv[c] - g_c
            sumsq = sumsq + d_c * d_c
        d_norm = jnp.sqrt(jnp.where(sumsq > 0, sumsq, 1.0))
        d_norm = jnp.where(sumsq > 0, d_norm, 0.0)
        dp = jnp.maximum(d_norm - _DELTA_AGG, 0.0)
        l = jnp.log(dp * dp + 1.0)
        loss = jnp.sum(l * tif) / jnp.maximum(tcnt, 1.0)
        valid = (kcnt > 0) & (tcnt > 0)
        loss_sum = loss_sum + jnp.where(valid, loss, 0.0)
        valid_cnt = valid_cnt + jnp.where(valid, 1.0, 0.0)

    final = jnp.where(valid_cnt > 0, loss_sum / jnp.maximum(valid_cnt, 1.0), 0.0)
    out_ref[0] = jnp.full((8, 128), final, jnp.float32)


def kernel(preds, targets):
    n = preds.shape[0]
    sim = preds[:, 2:, :, :]
    tt = targets[:, 0, :, :]
    tk = targets[:, 1, :, :]
    out = pl.pallas_call(
        _body,
        grid=(n,),
        in_specs=[
            pl.BlockSpec((1, 4, 128, 128), lambda i: (i, 0, 0, 0)),
            pl.BlockSpec((1, 128, 128), lambda i: (i, 0, 0)),
            pl.BlockSpec((1, 128, 128), lambda i: (i, 0, 0)),
        ],
        out_specs=pl.BlockSpec((1, 8, 128), lambda i: (i, 0, 0)),
        out_shape=jax.ShapeDtypeStruct((n, 8, 128), jnp.float32),
    )(sim, tt, tk)
    return out[:, 0, 0]
